# baseline (speedup 1.0000x reference)
import functools

import jax
import jax.numpy as jnp
from jax import lax
from jax.experimental import pallas as pl
from jax.experimental.pallas import tpu as pltpu
from jax.experimental.pallas import tpu_sc as plsc

B = 16384
D = 32
OUT_COLS = 4 * D + 1

NC = 2
NS = 16
NW = NC * NS
ROWS_PER_W = B // NW
CHUNK = 128
NCHUNK = ROWS_PER_W // CHUNK


def _sc_body(uid_hbm, zip_hbm, gen_hbm, age_hbm,
             emb_uid, emb_zip, emb_gen, emb_age,
             mean_hbm, inv_hbm, out_hbm,
             idx_v, ebuf, cbuf, stat_v, sem):
  wid = lax.axis_index("s") * NC + lax.axis_index("c")
  base = wid * ROWS_PER_W

  pltpu.sync_copy(mean_hbm, stat_v.at[0])
  pltpu.sync_copy(inv_hbm, stat_v.at[1])
  mean = stat_v[0, :]
  inv = stat_v[1, :]

  tables = (emb_uid, emb_zip, emb_gen, emb_age)
  ids = (uid_hbm, zip_hbm, gen_hbm, age_hbm)
  for f in range(4):
    pltpu.sync_copy(ids[f].at[pl.ds(wid * NCHUNK, NCHUNK)], idx_v.at[f])
    cps = []
    for j in range(NCHUNK):
      cps.append(pltpu.async_copy(
          tables[f].at[idx_v.at[f, j]],
          ebuf.at[pl.ds(j * CHUNK, CHUNK)], sem))
    for cp in cps:
      cp.wait()
    pltpu.sync_copy(ebuf, out_hbm.at[pl.ds(base, ROWS_PER_W),
                                     pl.ds(f * D, D)])

  zero = jnp.zeros((16,), jnp.int32)
  for i in range(ROWS_PER_W // 16):
    j, off = divmod(i * 16, CHUNK)
    a = idx_v[3, j, pl.ds(off, 16)]
    c = (a.astype(jnp.float32) - mean) * inv
    rows = lax.iota(jnp.int32, 16) + i * 16
    plsc.store_scatter(cbuf, [rows, zero], c)
  pltpu.sync_copy(cbuf, out_hbm.at[pl.ds(base, ROWS_PER_W),
                                   pl.ds(4 * D, 1)])


@jax.jit
def _run(uid2, zip2, gen2, age2, emb_uid, emb_zip, emb_gen, emb_age,
         mean16, inv16):
  mesh = plsc.VectorSubcoreMesh(core_axis_name="c", subcore_axis_name="s")
  return pl.kernel(
      _sc_body,
      out_type=jax.ShapeDtypeStruct((B, OUT_COLS), jnp.float32),
      mesh=mesh,
      scratch_types=[
          pltpu.VMEM((4, NCHUNK, CHUNK), jnp.int32),
          pltpu.VMEM((ROWS_PER_W, D), jnp.float32),
          pltpu.VMEM((ROWS_PER_W, 1), jnp.float32),
          pltpu.VMEM((2, 16), jnp.float32),
          pltpu.SemaphoreType.DMA,
      ],
      compiler_params=pltpu.CompilerParams(use_tc_tiling_on_sc=False,
                                           needs_layout_passes=False),
  )(uid2, zip2, gen2, age2, emb_uid, emb_zip, emb_gen, emb_age,
    mean16, inv16)


def kernel(user_id, user_zip_code, user_gender, bucketized_user_age,
           emb_user_id, emb_zip, emb_gender, emb_age, norm_mean, norm_var):
  inv_std = 1.0 / jnp.sqrt(norm_var + 1e-6)
  mean16 = jnp.broadcast_to(norm_mean, (16,))
  inv16 = jnp.broadcast_to(inv_std, (16,))
  shape2 = (NW * NCHUNK, CHUNK)
  return _run(user_id.reshape(shape2), user_zip_code.reshape(shape2),
              user_gender.reshape(shape2), bucketized_user_age.reshape(shape2),
              emb_user_id, emb_zip, emb_gender, emb_age, mean16, inv16)


# speedup vs baseline: 1.3251x; 1.3251x over previous
import functools

import jax
import jax.numpy as jnp
from jax import lax
from jax.experimental import pallas as pl
from jax.experimental.pallas import tpu as pltpu
from jax.experimental.pallas import tpu_sc as plsc

B = 16384
D = 32
GA_COLS = 2 * D
OUT_COLS = 4 * D + 1

NC = 2
NS = 16
NW = NC * NS
ROWS_PER_W = B // NW
CHUNK = 128
NCHUNK = ROWS_PER_W // CHUNK


def _sc_body(uid_hbm, zip_hbm, ga_hbm, mean_hbm, inv_hbm,
             emb_uid, emb_zip, emb_ga, out_hbm,
             idx_v, ebuf, gabuf, cbuf, stat_v, sem, wsem):
  wid = lax.axis_index("s") * NC + lax.axis_index("c")
  base = wid * ROWS_PER_W

  ids = (uid_hbm, zip_hbm, ga_hbm)
  for f in range(3):
    pltpu.sync_copy(ids[f].at[pl.ds(wid * NCHUNK, NCHUNK)], idx_v.at[f])
  cps = []
  for f, (tab, dst) in enumerate(((emb_uid, ebuf.at[0]),
                                  (emb_zip, ebuf.at[1]),
                                  (emb_ga, gabuf))):
    for j in range(NCHUNK):
      cps.append(pltpu.async_copy(
          tab.at[idx_v.at[f, j]],
          dst.at[pl.ds(j * CHUNK, CHUNK)], sem))

  pltpu.sync_copy(mean_hbm, stat_v.at[0])
  pltpu.sync_copy(inv_hbm, stat_v.at[1])
  mean = stat_v[0, :]
  inv = stat_v[1, :]
  zero = jnp.zeros((16,), jnp.int32)
  seven = jnp.full((16,), 7, jnp.int32)
  for i in range(ROWS_PER_W // 16):
    j, off = divmod(i * 16, CHUNK)
    a = lax.bitwise_and(idx_v[2, j, pl.ds(off, 16)], seven)
    c = (a.astype(jnp.float32) - mean) * inv
    rows16 = lax.iota(jnp.int32, 16) + i * 16
    plsc.store_scatter(cbuf, [rows16, zero], c)

  for cp in cps:
    cp.wait()

  rows = pl.ds(base, ROWS_PER_W)
  wps = [
      pltpu.async_copy(ebuf.at[0], out_hbm.at[rows, pl.ds(0, D)], wsem),
      pltpu.async_copy(ebuf.at[1], out_hbm.at[rows, pl.ds(D, D)], wsem),
      pltpu.async_copy(gabuf, out_hbm.at[rows, pl.ds(2 * D, GA_COLS)], wsem),
      pltpu.async_copy(cbuf, out_hbm.at[rows, pl.ds(4 * D, 1)], wsem),
  ]
  for wp in wps:
    wp.wait()


@jax.jit
def _run(uid2, zip2, ga2, mean16, inv16, emb_uid, emb_zip, emb_ga):
  mesh = plsc.VectorSubcoreMesh(core_axis_name="c", subcore_axis_name="s")
  return pl.kernel(
      _sc_body,
      out_type=jax.ShapeDtypeStruct((B, OUT_COLS), jnp.float32),
      mesh=mesh,
      scratch_types=[
          pltpu.VMEM((3, NCHUNK, CHUNK), jnp.int32),
          pltpu.VMEM((2, ROWS_PER_W, D), jnp.float32),
          pltpu.VMEM((ROWS_PER_W, GA_COLS), jnp.float32),
          pltpu.VMEM((ROWS_PER_W, 1), jnp.float32),
          pltpu.VMEM((2, 16), jnp.float32),
          pltpu.SemaphoreType.DMA,
          pltpu.SemaphoreType.DMA,
      ],
      compiler_params=pltpu.CompilerParams(use_tc_tiling_on_sc=False,
                                           needs_layout_passes=False),
  )(uid2, zip2, ga2, mean16, inv16, emb_uid, emb_zip, emb_ga)


def kernel(user_id, user_zip_code, user_gender, bucketized_user_age,
           emb_user_id, emb_zip, emb_gender, emb_age, norm_mean, norm_var):
  inv_std = 1.0 / jnp.sqrt(norm_var + 1e-6)
  n_age = emb_age.shape[0]
  n_gen = emb_gender.shape[0]
  emb_ga = jnp.concatenate([
      jnp.repeat(emb_gender, n_age, axis=0),
      jnp.tile(emb_age, (n_gen, 1)),
  ], axis=1)
  ga_idx = user_gender * n_age + bucketized_user_age
  mean16 = jnp.broadcast_to(norm_mean, (16,))
  inv16 = jnp.broadcast_to(inv_std, (16,))
  shape2 = (NW * NCHUNK, CHUNK)
  return _run(user_id.reshape(shape2), user_zip_code.reshape(shape2),
              ga_idx.reshape(shape2), mean16, inv16,
              emb_user_id, emb_zip, emb_ga)
